# Initial kernel scaffold; baseline (speedup 1.0000x reference)
import functools

import jax
import jax.numpy as jnp
from jax import lax
from jax.experimental import pallas as pl
from jax.experimental.pallas import tpu as pltpu
from jax.experimental.pallas import tpu_sc as plsc

N = 10000
E = 320000
D = 128

NC = 2
NS = 16
NW = NC * NS

E_PER_W = E // NW
CH = 80
NCHUNK = E_PER_W // CH

NPAD = 10240
DEG_PER_TILE = NPAD // NS

ROWS_PER_TILE = N // NS
ZROWS = 125


def _zero_1d(ref, nwords):
  zv = jnp.zeros((16,), jnp.float32)

  def body(i, _):
    ref[pl.ds(i * 16, 16)] = zv
    return 0

  lax.fori_loop(0, nwords // 16, body, 0)


def _zero_2d(ref, nrows):
  zv = jnp.zeros((16,), jnp.float32)

  def body(i, _):
    def inner(j, _):
      ref[i, pl.ds(j * 16, 16)] = zv
      return 0

    lax.fori_loop(0, 8, inner, 0)
    return 0

  lax.fori_loop(0, nrows, body, 0)


def _deg_body(edge_ref, out_ref, idx_v, ones_v, zb, dsrc_sh, ddst_sh):
  cid = lax.axis_index("c")
  sid = lax.axis_index("s")
  wid = sid * NC + cid

  ov = jnp.ones((16,), jnp.float32)
  for k in range(CH // 16):
    ones_v[pl.ds(k * 16, 16)] = ov

  _zero_1d(zb, DEG_PER_TILE)
  off = pl.multiple_of(sid * DEG_PER_TILE, 8)
  pltpu.sync_copy(zb, dsrc_sh.at[pl.ds(off, DEG_PER_TILE)])
  pltpu.sync_copy(zb, ddst_sh.at[pl.ds(off, DEG_PER_TILE)])
  plsc.subcore_barrier()

  def chunk(c, _):
    base = pl.multiple_of(wid * E_PER_W + c * CH, 8)
    pltpu.sync_copy(edge_ref.at[:, pl.ds(base, CH)], idx_v)
    pltpu.sync_copy(ones_v, dsrc_sh.at[idx_v.at[0]], add=True)
    pltpu.sync_copy(ones_v, ddst_sh.at[idx_v.at[1]], add=True)
    return 0

  lax.fori_loop(0, NCHUNK, chunk, 0)
  plsc.subcore_barrier()

  pltpu.sync_copy(dsrc_sh.at[pl.ds(off, DEG_PER_TILE)],
                  out_ref.at[cid, 0, pl.ds(off, DEG_PER_TILE)])
  pltpu.sync_copy(ddst_sh.at[pl.ds(off, DEG_PER_TILE)],
                  out_ref.at[cid, 1, pl.ds(off, DEG_PER_TILE)])


_deg_kernel = pl.kernel(
    _deg_body,
    out_type=jax.ShapeDtypeStruct((NC, 2, NPAD), jnp.float32),
    mesh=plsc.VectorSubcoreMesh(core_axis_name="c", subcore_axis_name="s"),
    scratch_types=[
        pltpu.VMEM((2, CH), jnp.int32),
        pltpu.VMEM((CH,), jnp.float32),
        pltpu.VMEM((DEG_PER_TILE,), jnp.float32),
        pltpu.VMEM_SHARED((NPAD,), jnp.float32),
        pltpu.VMEM_SHARED((NPAD,), jnp.float32),
    ],
)


def _scatter_body(p_ref, edge_ref, out_ref, idx_v, rows_v, zb, acc_sh):
  cid = lax.axis_index("c")
  sid = lax.axis_index("s")
  wid = sid * NC + cid

  _zero_2d(zb, ZROWS)
  for k in range(ROWS_PER_TILE // ZROWS):
    pltpu.sync_copy(zb, acc_sh.at[pl.ds(sid * ROWS_PER_TILE + k * ZROWS, ZROWS)])
  plsc.subcore_barrier()

  def chunk(c, _):
    base = pl.multiple_of(wid * E_PER_W + c * CH, 8)
    pltpu.sync_copy(edge_ref.at[:, pl.ds(base, CH)], idx_v)
    pltpu.sync_copy(p_ref.at[idx_v.at[0]], rows_v)
    pltpu.sync_copy(rows_v, acc_sh.at[idx_v.at[1]], add=True)
    return 0

  lax.fori_loop(0, NCHUNK, chunk, 0)
  plsc.subcore_barrier()

  for k in range(ROWS_PER_TILE // ZROWS):
    r0 = sid * ROWS_PER_TILE + k * ZROWS
    pltpu.sync_copy(acc_sh.at[pl.ds(r0, ZROWS)],
                    out_ref.at[cid, pl.ds(r0, ZROWS)])


_scatter_kernel = pl.kernel(
    _scatter_body,
    out_type=jax.ShapeDtypeStruct((NC, N, D), jnp.float32),
    mesh=plsc.VectorSubcoreMesh(core_axis_name="c", subcore_axis_name="s"),
    scratch_types=[
        pltpu.VMEM((2, CH), jnp.int32),
        pltpu.VMEM((CH, D), jnp.float32),
        pltpu.VMEM((ZROWS, D), jnp.float32),
        pltpu.VMEM_SHARED((N, D), jnp.float32),
    ],
)


def _dense_body(x_ref, w_ref, gamma_ref, beta_ref, deg_ref, p_ref):
  x = x_ref[...]
  mean = jnp.mean(x, axis=0)
  var = jnp.mean((x - mean) ** 2, axis=0)
  h = (x - mean) * lax.rsqrt(var + 1e-5) * gamma_ref[...] + beta_ref[...]
  h = jnp.maximum(h, 0.0)
  deg_src = deg_ref[0, 0, :] + deg_ref[1, 0, :]
  norm_src = jnp.where(deg_src > 0.0, lax.rsqrt(jnp.maximum(deg_src, 1.0)), 0.0)
  h = h * norm_src[:N, None]
  p_ref[...] = jnp.dot(h, w_ref[...], preferred_element_type=jnp.float32)


def _dense_kernel(x, W, gamma, beta, deg):
  return pl.pallas_call(
      _dense_body,
      out_shape=jax.ShapeDtypeStruct((N, D), jnp.float32),
  )(x, W, gamma, beta, deg)


def _final_body(x_ref, acc_ref, deg_ref, b_ref, out_ref):
  deg_dst = deg_ref[0, 1, :] + deg_ref[1, 1, :]
  norm_dst = jnp.where(deg_dst > 0.0, lax.rsqrt(jnp.maximum(deg_dst, 1.0)), 0.0)
  agg = acc_ref[0] + acc_ref[1]
  out_ref[...] = x_ref[...] + agg * norm_dst[:N, None] + b_ref[...]


def _final_kernel(x, acc, deg, b):
  return pl.pallas_call(
      _final_body,
      out_shape=jax.ShapeDtypeStruct((N, D), jnp.float32),
  )(x, acc, deg, b)


@jax.jit
def kernel(node_feats, edge_index, W, b, gamma, beta):
  ei = edge_index.astype(jnp.int32)
  deg = _deg_kernel(ei)
  p = _dense_kernel(node_feats, W, gamma, beta, deg)
  acc = _scatter_kernel(p, ei)
  return _final_kernel(node_feats, acc, deg, b)


# speedup vs baseline: 8.7467x; 8.7467x over previous
import functools

import jax
import jax.numpy as jnp
from jax import lax
from jax.experimental import pallas as pl
from jax.experimental.pallas import tpu as pltpu
from jax.experimental.pallas import tpu_sc as plsc

N = 10000
E = 320000
D = 128

NC = 2
NS = 16
NW = NC * NS

E_PER_W = E // NW
CH = 80
NCHUNK = E_PER_W // CH

NPAD = 10240
DEG_PER_TILE = NPAD // NS

ROWS_PER_TILE = NPAD // NS
ZROWS = 128


def _zero_1d(ref, nwords):
  zv = jnp.zeros((16,), jnp.float32)

  def body(i, _):
    ref[pl.ds(i * 16, 16)] = zv
    return 0

  lax.fori_loop(0, nwords // 16, body, 0)


def _zero_2d(ref, nrows):
  zv = jnp.zeros((16,), jnp.float32)

  def body(i, _):
    def inner(j, _):
      ref[i, pl.ds(j * 16, 16)] = zv
      return 0

    lax.fori_loop(0, 8, inner, 0)
    return 0

  lax.fori_loop(0, nrows, body, 0)


def _deg_body(src_ref, dst_ref, out_ref, idx_v, ones_v, zb, dsrc_sh, ddst_sh):
  cid = lax.axis_index("c")
  sid = lax.axis_index("s")
  wid = sid * NC + cid

  ov = jnp.ones((16,), jnp.float32)
  for k in range(CH // 16):
    ones_v[pl.ds(k * 16, 16)] = ov

  _zero_1d(zb, DEG_PER_TILE)
  off = pl.multiple_of(sid * DEG_PER_TILE, 8)
  pltpu.sync_copy(zb, dsrc_sh.at[pl.ds(off, DEG_PER_TILE)])
  pltpu.sync_copy(zb, ddst_sh.at[pl.ds(off, DEG_PER_TILE)])
  plsc.subcore_barrier()

  def chunk(c, _):
    base = pl.multiple_of(wid * E_PER_W + c * CH, 8)
    pltpu.sync_copy(src_ref.at[pl.ds(base, CH)], idx_v.at[0])
    pltpu.sync_copy(dst_ref.at[pl.ds(base, CH)], idx_v.at[1])
    pltpu.sync_copy(ones_v, dsrc_sh.at[idx_v.at[0]], add=True)
    pltpu.sync_copy(ones_v, ddst_sh.at[idx_v.at[1]], add=True)
    return 0

  lax.fori_loop(0, NCHUNK, chunk, 0)
  plsc.subcore_barrier()

  pltpu.sync_copy(dsrc_sh.at[pl.ds(off, DEG_PER_TILE)],
                  out_ref.at[cid, 0, pl.ds(off, DEG_PER_TILE)])
  pltpu.sync_copy(ddst_sh.at[pl.ds(off, DEG_PER_TILE)],
                  out_ref.at[cid, 1, pl.ds(off, DEG_PER_TILE)])


_deg_kernel = pl.kernel(
    _deg_body,
    out_type=jax.ShapeDtypeStruct((NC, 2, NPAD), jnp.float32),
    mesh=plsc.VectorSubcoreMesh(core_axis_name="c", subcore_axis_name="s"),
    scratch_types=[
        pltpu.VMEM((2, CH), jnp.int32),
        pltpu.VMEM((CH,), jnp.float32),
        pltpu.VMEM((DEG_PER_TILE,), jnp.float32),
        pltpu.VMEM_SHARED((NPAD,), jnp.float32),
        pltpu.VMEM_SHARED((NPAD,), jnp.float32),
    ],
)


def _scatter_body(p_ref, src_ref, dst_ref, out_ref, idx_v, rows_v, zb, acc_sh):
  cid = lax.axis_index("c")
  sid = lax.axis_index("s")
  wid = sid * NC + cid

  _zero_2d(zb, ZROWS)
  for k in range(ROWS_PER_TILE // ZROWS):
    r0 = pl.multiple_of(sid * ROWS_PER_TILE + k * ZROWS, 8)
    pltpu.sync_copy(zb, acc_sh.at[pl.ds(r0, ZROWS)])
  plsc.subcore_barrier()

  def chunk(c, _):
    base = pl.multiple_of(wid * E_PER_W + c * CH, 8)
    pltpu.sync_copy(src_ref.at[pl.ds(base, CH)], idx_v.at[0])
    pltpu.sync_copy(dst_ref.at[pl.ds(base, CH)], idx_v.at[1])
    pltpu.sync_copy(p_ref.at[idx_v.at[0]], rows_v)
    pltpu.sync_copy(rows_v, acc_sh.at[idx_v.at[1]], add=True)
    return 0

  lax.fori_loop(0, NCHUNK, chunk, 0)
  plsc.subcore_barrier()

  for k in range(ROWS_PER_TILE // ZROWS):
    r0 = pl.multiple_of(sid * ROWS_PER_TILE + k * ZROWS, 8)
    pltpu.sync_copy(acc_sh.at[pl.ds(r0, ZROWS)],
                    out_ref.at[cid, pl.ds(r0, ZROWS)])


_scatter_kernel = pl.kernel(
    _scatter_body,
    out_type=jax.ShapeDtypeStruct((NC, NPAD, D), jnp.float32),
    mesh=plsc.VectorSubcoreMesh(core_axis_name="c", subcore_axis_name="s"),
    scratch_types=[
        pltpu.VMEM((2, CH), jnp.int32),
        pltpu.VMEM((CH, D), jnp.float32),
        pltpu.VMEM((ZROWS, D), jnp.float32),
        pltpu.VMEM_SHARED((NPAD, D), jnp.float32),
    ],
)


def _dense_body(x_ref, w_ref, gamma_ref, beta_ref, deg_ref, p_ref):
  x = x_ref[...]
  mean = jnp.mean(x, axis=0)
  var = jnp.mean((x - mean) ** 2, axis=0)
  h = (x - mean) * lax.rsqrt(var + 1e-5) * gamma_ref[...] + beta_ref[...]
  h = jnp.maximum(h, 0.0)
  deg_src = deg_ref[0, 0, :] + deg_ref[1, 0, :]
  norm_src = jnp.where(deg_src > 0.0, lax.rsqrt(jnp.maximum(deg_src, 1.0)), 0.0)
  h = h * norm_src[:N, None]
  p_ref[...] = jnp.dot(h, w_ref[...], preferred_element_type=jnp.float32)


def _dense_kernel(x, W, gamma, beta, deg):
  return pl.pallas_call(
      _dense_body,
      out_shape=jax.ShapeDtypeStruct((N, D), jnp.float32),
  )(x, W, gamma, beta, deg)


def _final_body(x_ref, acc_ref, deg_ref, b_ref, out_ref):
  deg_dst = deg_ref[0, 1, :] + deg_ref[1, 1, :]
  norm_dst = jnp.where(deg_dst > 0.0, lax.rsqrt(jnp.maximum(deg_dst, 1.0)), 0.0)
  agg = acc_ref[0, :N] + acc_ref[1, :N]
  out_ref[...] = x_ref[...] + agg * norm_dst[:N, None] + b_ref[...]


def _final_kernel(x, acc, deg, b):
  return pl.pallas_call(
      _final_body,
      out_shape=jax.ShapeDtypeStruct((N, D), jnp.float32),
  )(x, acc, deg, b)


@jax.jit
def kernel(node_feats, edge_index, W, b, gamma, beta):
  ei = edge_index.astype(jnp.int32)
  src = ei[0]
  dst = ei[1]
  deg = _deg_kernel(src, dst)
  p = _dense_kernel(node_feats, W, gamma, beta, deg)
  acc = _scatter_kernel(p, src, dst)
  return _final_kernel(node_feats, acc, deg, b)
